# Initial kernel scaffold; baseline (speedup 1.0000x reference)
import jax
import jax.numpy as jnp
from jax.experimental import pallas as pl


def kernel(x, edge_index, edge_weight, W1, b1, W2, b2):
    raise NotImplementedError("write your pallas kernel here")


# speedup vs baseline: 16.9768x; 16.9768x over previous
import functools

import jax
import jax.numpy as jnp
from jax import lax
from jax.experimental import pallas as pl
from jax.experimental.pallas import tpu as pltpu
from jax.experimental.pallas import tpu_sc as plsc

N = 10000
E = 320000
D_IN = 128
D_H = 32
D_OUT = 16

L = 16
NC = 2
NS = 16
NW = NC * NS

EPT = E // NW
NZ = N // L

_sc_mesh = plsc.VectorSubcoreMesh(core_axis_name="c", subcore_axis_name="s")
_sc_params = pltpu.CompilerParams(needs_layout_passes=False)


def _wid():
    return lax.axis_index("s") * NC + lax.axis_index("c")


def _zero(ref):
    def body(i, carry):
        ref[pl.ds(i * L, L)] = jnp.zeros((L,), jnp.float32)
        return carry
    lax.fori_loop(0, NZ, body, None)


@functools.partial(
    pl.kernel,
    out_type=jax.ShapeDtypeStruct((NW, N), jnp.float32),
    mesh=_sc_mesh,
    compiler_params=_sc_params,
    scratch_types=[
        pltpu.VMEM((EPT,), jnp.int32),
        pltpu.VMEM((EPT,), jnp.float32),
        pltpu.VMEM((N,), jnp.float32),
    ],
)
def _sc_deg(col_hbm, ew_hbm, out_hbm, colv, ewv, degv):
    wid = _wid()
    _zero(degv)
    base = pl.multiple_of(wid * EPT, 8)
    pltpu.sync_copy(col_hbm.at[pl.ds(base, EPT)], colv)
    pltpu.sync_copy(ew_hbm.at[pl.ds(base, EPT)], ewv)

    def body(i, carry):
        sl = pl.ds(i * L, L)
        plsc.addupdate_scatter(degv, [colv[sl]], ewv[sl])
        return carry
    lax.fori_loop(0, EPT // L, body, None)
    pltpu.sync_copy(degv, out_hbm.at[wid])


@functools.partial(
    pl.kernel,
    out_type=jax.ShapeDtypeStruct((E,), jnp.float32),
    mesh=_sc_mesh,
    compiler_params=_sc_params,
    scratch_types=[
        pltpu.VMEM((EPT,), jnp.int32),
        pltpu.VMEM((EPT,), jnp.int32),
        pltpu.VMEM((EPT,), jnp.float32),
        pltpu.VMEM((EPT,), jnp.float32),
        pltpu.VMEM((N,), jnp.float32),
    ],
)
def _sc_norm(row_hbm, col_hbm, ew_hbm, dinv_hbm, out_hbm,
             rowv, colv, ewv, normv, dinvv):
    wid = _wid()
    pltpu.sync_copy(dinv_hbm.at[0], dinvv)
    base = pl.multiple_of(wid * EPT, 8)
    pltpu.sync_copy(row_hbm.at[pl.ds(base, EPT)], rowv)
    pltpu.sync_copy(col_hbm.at[pl.ds(base, EPT)], colv)
    pltpu.sync_copy(ew_hbm.at[pl.ds(base, EPT)], ewv)

    def body(i, carry):
        sl = pl.ds(i * L, L)
        dr = plsc.load_gather(dinvv, [rowv[sl]])
        dc = plsc.load_gather(dinvv, [colv[sl]])
        normv[sl] = dr * ewv[sl] * dc
        return carry
    lax.fori_loop(0, EPT // L, body, None)
    pltpu.sync_copy(normv, out_hbm.at[pl.ds(base, EPT)])


def _make_sc_agg(NF):
    TPF = NW // NF
    EPP = E // TPF
    CH = 10000
    NCHK = EPP // CH

    @functools.partial(
        pl.kernel,
        out_type=jax.ShapeDtypeStruct((NW, N), jnp.float32),
        mesh=_sc_mesh,
    compiler_params=_sc_params,
        scratch_types=[
            pltpu.VMEM((CH,), jnp.int32),
            pltpu.VMEM((CH,), jnp.int32),
            pltpu.VMEM((CH,), jnp.float32),
            pltpu.VMEM((N,), jnp.float32),
            pltpu.VMEM((N,), jnp.float32),
        ],
    )
    def _sc_agg(row_hbm, col_hbm, norm_hbm, ht_hbm, out_hbm,
                rowv, colv, normv, hfv, accv):
        wid = _wid()
        f = wid % NF
        half = wid // NF
        pltpu.sync_copy(ht_hbm.at[f], hfv)
        _zero(accv)

        def chunk(j, carry):
            off = pl.multiple_of(half * EPP + j * CH, 8)
            pltpu.sync_copy(row_hbm.at[pl.ds(off, CH)], rowv)
            pltpu.sync_copy(col_hbm.at[pl.ds(off, CH)], colv)
            pltpu.sync_copy(norm_hbm.at[pl.ds(off, CH)], normv)

            def body(i, c2):
                sl = pl.ds(i * L, L)
                hv = plsc.load_gather(hfv, [rowv[sl]])
                plsc.addupdate_scatter(accv, [colv[sl]], normv[sl] * hv)
                return c2
            lax.fori_loop(0, CH // L, body, None)
            return carry
        lax.fori_loop(0, NCHK, chunk, None)
        pltpu.sync_copy(accv, out_hbm.at[wid])

    return _sc_agg


_sc_agg_l1 = _make_sc_agg(D_H)
_sc_agg_l2 = _make_sc_agg(D_OUT)


def _tc_dinv(degp):
    def body(degp_ref, dinv_ref):
        deg = jnp.sum(degp_ref[...], axis=0, keepdims=True) + 1.0
        safe = jnp.where(deg > 0, deg, 1.0)
        dinv_ref[...] = jnp.where(deg > 0, 1.0 / jnp.sqrt(safe), 0.0)
    return pl.pallas_call(
        body, out_shape=jax.ShapeDtypeStruct((1, N), jnp.float32))(degp)


def _tc_h1t(x, W1):
    def body(x_ref, w_ref, out_ref):
        out_ref[...] = lax.dot_general(
            w_ref[...], x_ref[...], (((0,), (1,)), ((), ())),
            preferred_element_type=jnp.float32)
    return pl.pallas_call(
        body, out_shape=jax.ShapeDtypeStruct((D_H, N), jnp.float32))(x, W1)


def _tc_mid(agg1, h1t, dinv, b1c, W2):
    def body(agg_ref, h_ref, dinv_ref, b_ref, w_ref, out_ref):
        dinv = dinv_ref[...]
        z = agg_ref[...] + dinv * dinv * h_ref[...] + b_ref[...]
        z = jnp.maximum(z, 0.0)
        out_ref[...] = lax.dot_general(
            w_ref[...], z, (((0,), (0,)), ((), ())),
            preferred_element_type=jnp.float32)
    return pl.pallas_call(
        body, out_shape=jax.ShapeDtypeStruct((D_OUT, N), jnp.float32))(
            agg1, h1t, dinv, b1c, W2)


def _tc_final(agg2p, h2t, dinv, b2c):
    def body(agg_ref, h_ref, dinv_ref, b_ref, out_ref):
        dinv = dinv_ref[...]
        s = agg_ref[0:D_OUT, :] + agg_ref[D_OUT:NW, :]
        out_ref[...] = s + dinv * dinv * h_ref[...] + b_ref[...]
    return pl.pallas_call(
        body, out_shape=jax.ShapeDtypeStruct((D_OUT, N), jnp.float32))(
            agg2p, h2t, dinv, b2c)


def kernel(x, edge_index, edge_weight, W1, b1, W2, b2):
    row = edge_index[0]
    col = edge_index[1]
    degp = _sc_deg(col, edge_weight)
    dinv = _tc_dinv(degp)
    norm = _sc_norm(row, col, edge_weight, dinv)
    h1t = _tc_h1t(x, W1)
    agg1 = _sc_agg_l1(row, col, norm, h1t)
    h2t = _tc_mid(agg1, h1t, dinv, b1.reshape(D_H, 1), W2)
    agg2p = _sc_agg_l2(row, col, norm, h2t)
    out16 = _tc_final(agg2p, h2t, dinv, b2.reshape(D_OUT, 1))
    return out16.T


# speedup vs baseline: 27.3269x; 1.6097x over previous
import functools

import jax
import jax.numpy as jnp
from jax import lax
from jax.experimental import pallas as pl
from jax.experimental.pallas import tpu as pltpu
from jax.experimental.pallas import tpu_sc as plsc

N = 10000
E = 320000
D_IN = 128
D_H = 32
D_OUT = 16

L = 16
NC = 2
NS = 16
NW = NC * NS

EPT = E // NW
NZ = N // L

_sc_mesh = plsc.VectorSubcoreMesh(core_axis_name="c", subcore_axis_name="s")
_sc_params = pltpu.CompilerParams(needs_layout_passes=False)


def _wid():
    return lax.axis_index("s") * NC + lax.axis_index("c")


def _zero(ref):
    def body(i, carry):
        ref[pl.ds(i * L, L)] = jnp.zeros((L,), jnp.float32)
        return carry
    lax.fori_loop(0, NZ, body, None)


@functools.partial(
    pl.kernel,
    out_type=jax.ShapeDtypeStruct((NW, N), jnp.float32),
    mesh=_sc_mesh,
    compiler_params=_sc_params,
    scratch_types=[
        pltpu.VMEM((EPT,), jnp.int32),
        pltpu.VMEM((EPT,), jnp.float32),
        pltpu.VMEM((N,), jnp.float32),
    ],
)
def _sc_deg(col_hbm, ew_hbm, out_hbm, colv, ewv, degv):
    wid = _wid()
    _zero(degv)
    base = pl.multiple_of(wid * EPT, 8)
    pltpu.sync_copy(col_hbm.at[pl.ds(base, EPT)], colv)
    pltpu.sync_copy(ew_hbm.at[pl.ds(base, EPT)], ewv)

    def body(i, carry):
        sl = pl.ds(i * L, L)
        plsc.addupdate_scatter(degv, [colv[sl]], ewv[sl])
        return carry
    lax.fori_loop(0, EPT // L, body, None)
    pltpu.sync_copy(degv, out_hbm.at[wid])


@functools.partial(
    pl.kernel,
    out_type=jax.ShapeDtypeStruct((E,), jnp.float32),
    mesh=_sc_mesh,
    compiler_params=_sc_params,
    scratch_types=[
        pltpu.VMEM((EPT,), jnp.int32),
        pltpu.VMEM((EPT,), jnp.int32),
        pltpu.VMEM((EPT,), jnp.float32),
        pltpu.VMEM((EPT,), jnp.float32),
        pltpu.VMEM((N,), jnp.float32),
    ],
)
def _sc_norm(row_hbm, col_hbm, ew_hbm, dinv_hbm, out_hbm,
             rowv, colv, ewv, normv, dinvv):
    wid = _wid()
    pltpu.sync_copy(dinv_hbm.at[0], dinvv)
    base = pl.multiple_of(wid * EPT, 8)
    pltpu.sync_copy(row_hbm.at[pl.ds(base, EPT)], rowv)
    pltpu.sync_copy(col_hbm.at[pl.ds(base, EPT)], colv)
    pltpu.sync_copy(ew_hbm.at[pl.ds(base, EPT)], ewv)

    def body(i, carry):
        sl = pl.ds(i * L, L)
        dr = plsc.load_gather(dinvv, [rowv[sl]])
        dc = plsc.load_gather(dinvv, [colv[sl]])
        normv[sl] = dr * ewv[sl] * dc
        return carry
    lax.fori_loop(0, EPT // L, body, None)
    pltpu.sync_copy(normv, out_hbm.at[pl.ds(base, EPT)])


FPT = 4
UNR = 4


def _make_sc_agg(NF):
    NG = NF // FPT
    TPG = NW // NG
    EPP = E // TPG
    CH = 8000
    NCHK = EPP // CH

    @functools.partial(
        pl.kernel,
        out_type=jax.ShapeDtypeStruct((TPG, NF, N), jnp.float32),
        mesh=_sc_mesh,
        compiler_params=_sc_params,
        scratch_types=[
            pltpu.VMEM((CH,), jnp.int32),
            pltpu.VMEM((CH,), jnp.int32),
            pltpu.VMEM((CH,), jnp.float32),
        ] + [pltpu.VMEM((N,), jnp.float32) for _ in range(2 * FPT)],
    )
    def _sc_agg(row_hbm, col_hbm, norm_hbm, ht_hbm, out_hbm,
                rowv, colv, normv, *hf_acc):
        hf = hf_acc[:FPT]
        acc = hf_acc[FPT:]
        wid = _wid()
        g = wid % NG
        sub = wid // NG
        for k in range(FPT):
            pltpu.sync_copy(ht_hbm.at[g * FPT + k], hf[k])

        def zbody(i, carry):
            z = jnp.zeros((L,), jnp.float32)
            for k in range(FPT):
                acc[k][pl.ds(i * L, L)] = z
            return carry
        lax.fori_loop(0, NZ, zbody, None)

        def chunk(j, carry):
            off = pl.multiple_of(sub * EPP + j * CH, 8)
            pltpu.sync_copy(row_hbm.at[pl.ds(off, CH)], rowv)
            pltpu.sync_copy(col_hbm.at[pl.ds(off, CH)], colv)
            pltpu.sync_copy(norm_hbm.at[pl.ds(off, CH)], normv)

            def body(i, c2):
                for u in range(UNR):
                    sl = pl.ds((i * UNR + u) * L, L)
                    r = rowv[sl]
                    c = colv[sl]
                    nv = normv[sl]
                    for k in range(FPT):
                        hv = plsc.load_gather(hf[k], [r])
                        plsc.addupdate_scatter(acc[k], [c], nv * hv)
                return c2
            lax.fori_loop(0, CH // (L * UNR), body, None)
            return carry
        lax.fori_loop(0, NCHK, chunk, None)
        for k in range(FPT):
            pltpu.sync_copy(acc[k], out_hbm.at[sub, g * FPT + k])

    return _sc_agg


_sc_agg_l1 = _make_sc_agg(D_H)
_sc_agg_l2 = _make_sc_agg(D_OUT)
TPG1 = NW // (D_H // FPT)
TPG2 = NW // (D_OUT // FPT)


def _tc_dinv(degp):
    def body(degp_ref, dinv_ref):
        deg = jnp.sum(degp_ref[...], axis=0, keepdims=True) + 1.0
        safe = jnp.where(deg > 0, deg, 1.0)
        dinv_ref[...] = jnp.where(deg > 0, 1.0 / jnp.sqrt(safe), 0.0)
    return pl.pallas_call(
        body, out_shape=jax.ShapeDtypeStruct((1, N), jnp.float32))(degp)


def _tc_h1t(x, W1):
    def body(x_ref, w_ref, out_ref):
        out_ref[...] = lax.dot_general(
            w_ref[...], x_ref[...], (((0,), (1,)), ((), ())),
            preferred_element_type=jnp.float32)
    return pl.pallas_call(
        body, out_shape=jax.ShapeDtypeStruct((D_H, N), jnp.float32))(x, W1)


def _tc_mid(agg1, h1t, dinv, b1c, W2):
    def body(agg_ref, h_ref, dinv_ref, b_ref, w_ref, out_ref):
        dinv = dinv_ref[...]
        s = agg_ref[0]
        for p in range(1, TPG1):
            s = s + agg_ref[p]
        z = s + dinv * dinv * h_ref[...] + b_ref[...]
        z = jnp.maximum(z, 0.0)
        out_ref[...] = lax.dot_general(
            w_ref[...], z, (((0,), (0,)), ((), ())),
            preferred_element_type=jnp.float32)
    return pl.pallas_call(
        body, out_shape=jax.ShapeDtypeStruct((D_OUT, N), jnp.float32))(
            agg1, h1t, dinv, b1c, W2)


def _tc_final(agg2p, h2t, dinv, b2c):
    def body(agg_ref, h_ref, dinv_ref, b_ref, out_ref):
        dinv = dinv_ref[...]
        s = agg_ref[0]
        for p in range(1, TPG2):
            s = s + agg_ref[p]
        out_ref[...] = s + dinv * dinv * h_ref[...] + b_ref[...]
    return pl.pallas_call(
        body, out_shape=jax.ShapeDtypeStruct((D_OUT, N), jnp.float32))(
            agg2p, h2t, dinv, b2c)


def kernel(x, edge_index, edge_weight, W1, b1, W2, b2):
    row = edge_index[0]
    col = edge_index[1]
    degp = _sc_deg(col, edge_weight)
    dinv = _tc_dinv(degp)
    norm = _sc_norm(row, col, edge_weight, dinv)
    h1t = _tc_h1t(x, W1)
    agg1 = _sc_agg_l1(row, col, norm, h1t)
    h2t = _tc_mid(agg1, h1t, dinv, b1.reshape(D_H, 1), W2)
    agg2p = _sc_agg_l2(row, col, norm, h2t)
    out16 = _tc_final(agg2p, h2t, dinv, b2.reshape(D_OUT, 1))
    return out16.T


# speedup vs baseline: 49.6026x; 1.8152x over previous
import functools

import jax
import jax.numpy as jnp
from jax import lax
from jax.experimental import pallas as pl
from jax.experimental.pallas import tpu as pltpu
from jax.experimental.pallas import tpu_sc as plsc

N = 10000
E = 320000
D_IN = 128
D_H = 32
D_OUT = 16

L = 16
NC = 2
NS = 16
NW = NC * NS

EPT = E // NW
NZ = N // L

_sc_mesh = plsc.VectorSubcoreMesh(core_axis_name="c", subcore_axis_name="s")
_sc_params = pltpu.CompilerParams(needs_layout_passes=False)


def _wid():
    return lax.axis_index("s") * NC + lax.axis_index("c")


def _zero(ref):
    @plsc.parallel_loop(0, NZ, unroll=5)
    def body(i):
        ref[pl.ds(i * L, L)] = jnp.zeros((L,), jnp.float32)


@functools.partial(
    pl.kernel,
    out_type=jax.ShapeDtypeStruct((NW, N), jnp.float32),
    mesh=_sc_mesh,
    compiler_params=_sc_params,
    scratch_types=[
        pltpu.VMEM((EPT,), jnp.int32),
        pltpu.VMEM((EPT,), jnp.float32),
        pltpu.VMEM((N,), jnp.float32),
    ],
)
def _sc_deg(col_hbm, ew_hbm, out_hbm, colv, ewv, degv):
    wid = _wid()
    _zero(degv)
    base = pl.multiple_of(wid * EPT, 8)
    pltpu.sync_copy(col_hbm.at[pl.ds(base, EPT)], colv)
    pltpu.sync_copy(ew_hbm.at[pl.ds(base, EPT)], ewv)

    @plsc.parallel_loop(0, EPT // L, unroll=5)
    def body(i):
        sl = pl.ds(i * L, L)
        plsc.addupdate_scatter(degv, [colv[sl]], ewv[sl])
    pltpu.sync_copy(degv, out_hbm.at[wid])


@functools.partial(
    pl.kernel,
    out_type=jax.ShapeDtypeStruct((E,), jnp.float32),
    mesh=_sc_mesh,
    compiler_params=_sc_params,
    scratch_types=[
        pltpu.VMEM((EPT,), jnp.int32),
        pltpu.VMEM((EPT,), jnp.int32),
        pltpu.VMEM((EPT,), jnp.float32),
        pltpu.VMEM((EPT,), jnp.float32),
        pltpu.VMEM((N,), jnp.float32),
    ],
)
def _sc_norm(row_hbm, col_hbm, ew_hbm, dinv_hbm, out_hbm,
             rowv, colv, ewv, normv, dinvv):
    wid = _wid()
    pltpu.sync_copy(dinv_hbm.at[0], dinvv)
    base = pl.multiple_of(wid * EPT, 8)
    pltpu.sync_copy(row_hbm.at[pl.ds(base, EPT)], rowv)
    pltpu.sync_copy(col_hbm.at[pl.ds(base, EPT)], colv)
    pltpu.sync_copy(ew_hbm.at[pl.ds(base, EPT)], ewv)

    @plsc.parallel_loop(0, EPT // L, unroll=5)
    def body(i):
        sl = pl.ds(i * L, L)
        dr = plsc.load_gather(dinvv, [rowv[sl]])
        dc = plsc.load_gather(dinvv, [colv[sl]])
        normv[sl] = dr * ewv[sl] * dc
    pltpu.sync_copy(normv, out_hbm.at[pl.ds(base, EPT)])


FPT = 4


def _make_sc_agg(NF):
    NG = NF // FPT
    TPG = NW // NG
    EPP = E // TPG
    CH = 10000
    NCHK = EPP // CH

    @functools.partial(
        pl.kernel,
        out_type=jax.ShapeDtypeStruct((TPG, NF, N), jnp.float32),
        mesh=_sc_mesh,
        compiler_params=_sc_params,
        scratch_types=[
            pltpu.VMEM((CH,), jnp.int32),
            pltpu.VMEM((CH,), jnp.int32),
            pltpu.VMEM((CH,), jnp.float32),
        ] + [pltpu.VMEM((N,), jnp.float32) for _ in range(2 * FPT)],
    )
    def _sc_agg(row_hbm, col_hbm, norm_hbm, ht_hbm, out_hbm,
                rowv, colv, normv, *hf_acc):
        hf = hf_acc[:FPT]
        acc = hf_acc[FPT:]
        wid = _wid()
        g = wid % NG
        sub = wid // NG
        for k in range(FPT):
            pltpu.sync_copy(ht_hbm.at[g * FPT + k], hf[k])

        @plsc.parallel_loop(0, NZ, unroll=5)
        def zbody(i):
            z = jnp.zeros((L,), jnp.float32)
            for k in range(FPT):
                acc[k][pl.ds(i * L, L)] = z

        def chunk(j, carry):
            off = pl.multiple_of(sub * EPP + j * CH, 8)
            pltpu.sync_copy(row_hbm.at[pl.ds(off, CH)], rowv)
            pltpu.sync_copy(col_hbm.at[pl.ds(off, CH)], colv)
            pltpu.sync_copy(norm_hbm.at[pl.ds(off, CH)], normv)

            @plsc.parallel_loop(0, CH // L, unroll=5)
            def body(i):
                sl = pl.ds(i * L, L)
                r = rowv[sl]
                c = colv[sl]
                nv = normv[sl]
                for k in range(FPT):
                    hv = plsc.load_gather(hf[k], [r])
                    plsc.addupdate_scatter(acc[k], [c], nv * hv)
            return carry
        lax.fori_loop(0, NCHK, chunk, None)
        for k in range(FPT):
            pltpu.sync_copy(acc[k], out_hbm.at[sub, g * FPT + k])

    return _sc_agg


_sc_agg_l1 = _make_sc_agg(D_H)
_sc_agg_l2 = _make_sc_agg(D_OUT)
TPG1 = NW // (D_H // FPT)
TPG2 = NW // (D_OUT // FPT)


def _tc_dinv(degp):
    def body(degp_ref, dinv_ref):
        deg = jnp.sum(degp_ref[...], axis=0, keepdims=True) + 1.0
        safe = jnp.where(deg > 0, deg, 1.0)
        dinv_ref[...] = jnp.where(deg > 0, 1.0 / jnp.sqrt(safe), 0.0)
    return pl.pallas_call(
        body, out_shape=jax.ShapeDtypeStruct((1, N), jnp.float32))(degp)


def _tc_h1t(x, W1):
    def body(x_ref, w_ref, out_ref):
        out_ref[...] = lax.dot_general(
            w_ref[...], x_ref[...], (((0,), (1,)), ((), ())),
            preferred_element_type=jnp.float32)
    return pl.pallas_call(
        body, out_shape=jax.ShapeDtypeStruct((D_H, N), jnp.float32))(x, W1)


def _tc_mid(agg1, h1t, dinv, b1c, W2):
    def body(agg_ref, h_ref, dinv_ref, b_ref, w_ref, out_ref):
        dinv = dinv_ref[...]
        s = agg_ref[0]
        for p in range(1, TPG1):
            s = s + agg_ref[p]
        z = s + dinv * dinv * h_ref[...] + b_ref[...]
        z = jnp.maximum(z, 0.0)
        out_ref[...] = lax.dot_general(
            w_ref[...], z, (((0,), (0,)), ((), ())),
            preferred_element_type=jnp.float32)
    return pl.pallas_call(
        body, out_shape=jax.ShapeDtypeStruct((D_OUT, N), jnp.float32))(
            agg1, h1t, dinv, b1c, W2)


def _tc_final(agg2p, h2t, dinv, b2c):
    def body(agg_ref, h_ref, dinv_ref, b_ref, out_ref):
        dinv = dinv_ref[...]
        s = agg_ref[0]
        for p in range(1, TPG2):
            s = s + agg_ref[p]
        out_ref[...] = s + dinv * dinv * h_ref[...] + b_ref[...]
    return pl.pallas_call(
        body, out_shape=jax.ShapeDtypeStruct((D_OUT, N), jnp.float32))(
            agg2p, h2t, dinv, b2c)


def kernel(x, edge_index, edge_weight, W1, b1, W2, b2):
    row = edge_index[0]
    col = edge_index[1]
    degp = _sc_deg(col, edge_weight)
    dinv = _tc_dinv(degp)
    norm = _sc_norm(row, col, edge_weight, dinv)
    h1t = _tc_h1t(x, W1)
    agg1 = _sc_agg_l1(row, col, norm, h1t)
    h2t = _tc_mid(agg1, h1t, dinv, b1.reshape(D_H, 1), W2)
    agg2p = _sc_agg_l2(row, col, norm, h2t)
    out16 = _tc_final(agg2p, h2t, dinv, b2.reshape(D_OUT, 1))
    return out16.T
